# baseline (speedup 1.0000x reference)
import functools

import jax
import jax.numpy as jnp
from jax import lax
from jax.experimental import pallas as pl
from jax.experimental.pallas import tpu as pltpu
from jax.experimental.pallas import tpu_sc as plsc

_BATCH = 4096
_HIST = 50
_EMBED = 64

_NC = 2
_NS = 16
_NW = _NC * _NS
_BB = _BATCH // _NW
_NB = 16
_NCHUNK = _BB // _NB
_NBUF = 2

_mesh = plsc.VectorSubcoreMesh(core_axis_name="c", subcore_axis_name="s")


@functools.partial(
    pl.kernel,
    mesh=_mesh,
    out_type=jax.ShapeDtypeStruct((_BATCH, _HIST, _EMBED), jnp.float32),
    compiler_params=pltpu.CompilerParams(use_tc_tiling_on_sc=False),
    scratch_types=[
        pltpu.VMEM((_BB, _HIST), jnp.int32),
        pltpu.VMEM((_NBUF, _NB, _HIST, _EMBED), jnp.float32),
        pltpu.SemaphoreType.DMA,
        pltpu.SemaphoreType.DMA,
        pltpu.SemaphoreType.DMA,
        pltpu.SemaphoreType.DMA,
    ],
)
def _gather(idx_hbm, table_hbm, out_hbm, idx_v, rows_v, g0, g1, w0, w1):
    wid = lax.axis_index("s") * _NC + lax.axis_index("c")
    bbase = wid * _BB
    pltpu.sync_copy(idx_hbm.at[pl.ds(bbase, _BB)], idx_v)

    gsem = (g0, g1)
    wsem = (w0, w1)
    gathers = [None] * _NBUF
    writes = [None] * _NBUF
    for c in range(_NCHUNK + 1):
        if c < _NCHUNK:
            buf = c % _NBUF
            if writes[buf] is not None:
                writes[buf].wait()
                writes[buf] = None
            gathers[buf] = [
                pltpu.async_copy(
                    table_hbm.at[idx_v.at[c * _NB + j]],
                    rows_v.at[buf, j],
                    gsem[buf],
                )
                for j in range(_NB)
            ]
        if c > 0:
            pbuf = (c - 1) % _NBUF
            for g in gathers[pbuf]:
                g.wait()
            writes[pbuf] = pltpu.async_copy(
                rows_v.at[pbuf],
                out_hbm.at[pl.ds(bbase + (c - 1) * _NB, _NB)],
                wsem[pbuf],
            )
    for buf in range(_NBUF):
        if writes[buf] is not None:
            writes[buf].wait()


def kernel(inputs, table):
    return _gather(inputs.astype(jnp.int32), table)
